# Initial kernel scaffold; baseline (speedup 1.0000x reference)
import jax
import jax.numpy as jnp
from jax.experimental import pallas as pl


def kernel(label, view_label, time_label, tokenized_table, token_embedding, ctx):
    raise NotImplementedError("write your pallas kernel here")


# speedup vs baseline: 4.5022x; 4.5022x over previous
import functools

import jax
import jax.numpy as jnp
from jax import lax
from jax.experimental import pallas as pl
from jax.experimental.pallas import tpu as pltpu
from jax.experimental.pallas import tpu_sc as plsc

SEQ = 77
DIM = 512
N_CTX = 4
X_POS = 5

ROWS_PAD = 80
CHUNK = 16
N_CHUNKS = ROWS_PAD // CHUNK
N_WORKERS = 4 * N_CHUNKS

BS = 8


def _sc_gather_prompts(idx_flat, table):
    info = plsc.get_sparse_core_info()
    nc = info.num_cores
    mesh = plsc.VectorSubcoreMesh(core_axis_name="c", subcore_axis_name="s")

    @functools.partial(
        pl.kernel,
        mesh=mesh,
        out_type=jax.ShapeDtypeStruct((4 * ROWS_PAD, DIM), jnp.float32),
        scratch_types=[
            pltpu.VMEM((CHUNK,), jnp.int32),
            pltpu.VMEM((CHUNK, DIM), jnp.float32),
            pltpu.SemaphoreType.DMA,
        ],
    )
    def k(idx_hbm, table_hbm, out_hbm, idx_v, rows_v, sem):
        wid = lax.axis_index("s") * nc + lax.axis_index("c")

        @pl.when(wid < N_WORKERS)
        def _():
            base = pl.multiple_of(wid * CHUNK, CHUNK)
            pltpu.sync_copy(idx_hbm.at[pl.ds(base, CHUNK)], idx_v)
            pltpu.async_copy(table_hbm.at[idx_v], rows_v, sem).wait()
            pltpu.sync_copy(rows_v, out_hbm.at[pl.ds(base, CHUNK)])

    return k(idx_flat, table)


def _tc_assemble(view, time, prompts, ctx77):
    B = view.shape[0]

    def body(view_ref, time_ref, prompts_ref, ctx_ref, out_ref):
        g = pl.program_id(0)
        row = lax.broadcasted_iota(jnp.int32, (SEQ, DIM), 0)
        is_ctx = (row >= X_POS) & (row < X_POS + N_CTX)
        ctx_rows = ctx_ref[...]
        for j in range(BS):
            b = g * BS + j
            c = view_ref[b] * 2 + time_ref[b]
            out_ref[j] = jnp.where(is_ctx, ctx_rows, prompts_ref[c][0:SEQ, :])

    return pl.pallas_call(
        body,
        grid=(B // BS,),
        in_specs=[
            pl.BlockSpec(memory_space=pltpu.SMEM),
            pl.BlockSpec(memory_space=pltpu.SMEM),
            pl.BlockSpec((4, ROWS_PAD, DIM), lambda g: (0, 0, 0)),
            pl.BlockSpec((SEQ, DIM), lambda g: (0, 0)),
        ],
        out_specs=pl.BlockSpec((BS, SEQ, DIM), lambda g: (g, 0, 0)),
        out_shape=jax.ShapeDtypeStruct((B, SEQ, DIM), jnp.float32),
    )(view, time, prompts, ctx77)


def kernel(label, view_label, time_label, tokenized_table, token_embedding, ctx):
    del label
    idx = jnp.pad(tokenized_table.astype(jnp.int32), ((0, 0), (0, ROWS_PAD - SEQ)))
    prompts = _sc_gather_prompts(idx.reshape(-1), token_embedding)
    ctx77 = jnp.pad(ctx, ((X_POS, SEQ - X_POS - N_CTX), (0, 0)))
    return _tc_assemble(
        view_label.astype(jnp.int32),
        time_label.astype(jnp.int32),
        prompts.reshape(4, ROWS_PAD, DIM),
        ctx77,
    )


# speedup vs baseline: 5.0244x; 1.1160x over previous
import functools

import jax
import jax.numpy as jnp
from jax import lax
from jax.experimental import pallas as pl
from jax.experimental.pallas import tpu as pltpu
from jax.experimental.pallas import tpu_sc as plsc

SEQ = 77
DIM = 512
N_CTX = 4
X_POS = 5

ROWS_PAD = 80
CHUNK = 16
N_CHUNKS = ROWS_PAD // CHUNK
N_WORKERS = 4 * N_CHUNKS

BS = 16


def _sc_gather_prompts(idx_flat, table):
    info = plsc.get_sparse_core_info()
    nc = info.num_cores
    mesh = plsc.VectorSubcoreMesh(core_axis_name="c", subcore_axis_name="s")

    @functools.partial(
        pl.kernel,
        mesh=mesh,
        out_type=jax.ShapeDtypeStruct((4 * ROWS_PAD, DIM), jnp.float32),
        scratch_types=[
            pltpu.VMEM((CHUNK,), jnp.int32),
            pltpu.VMEM((CHUNK, DIM), jnp.float32),
            pltpu.SemaphoreType.DMA,
        ],
    )
    def k(idx_hbm, table_hbm, out_hbm, idx_v, rows_v, sem):
        wid = lax.axis_index("s") * nc + lax.axis_index("c")

        @pl.when(wid < N_WORKERS)
        def _():
            base = pl.multiple_of(wid * CHUNK, CHUNK)
            pltpu.sync_copy(idx_hbm.at[pl.ds(base, CHUNK)], idx_v)
            pltpu.async_copy(table_hbm.at[idx_v], rows_v, sem).wait()
            pltpu.sync_copy(rows_v, out_hbm.at[pl.ds(base, CHUNK)])

    return k(idx_flat, table)


def _tc_assemble(view, time, prompts, ctx77):
    B = view.shape[0]

    def body(view_ref, time_ref, prompts_ref, ctx_ref, out_ref):
        g = pl.program_id(0)
        row = lax.broadcasted_iota(jnp.int32, (SEQ, DIM), 0)
        is_ctx = (row >= X_POS) & (row < X_POS + N_CTX)
        ctx_rows = ctx_ref[...]
        for j in range(BS):
            b = g * BS + j
            c = view_ref[b] * 2 + time_ref[b]
            out_ref[j] = jnp.where(is_ctx, ctx_rows, prompts_ref[c][0:SEQ, :])

    return pl.pallas_call(
        body,
        grid=(B // BS,),
        in_specs=[
            pl.BlockSpec(memory_space=pltpu.SMEM),
            pl.BlockSpec(memory_space=pltpu.SMEM),
            pl.BlockSpec((4, ROWS_PAD, DIM), lambda g: (0, 0, 0)),
            pl.BlockSpec((SEQ, DIM), lambda g: (0, 0)),
        ],
        out_specs=pl.BlockSpec((BS, SEQ, DIM), lambda g: (g, 0, 0)),
        out_shape=jax.ShapeDtypeStruct((B, SEQ, DIM), jnp.float32),
    )(view, time, prompts, ctx77)


def kernel(label, view_label, time_label, tokenized_table, token_embedding, ctx):
    del label
    idx = jnp.pad(tokenized_table.astype(jnp.int32), ((0, 0), (0, ROWS_PAD - SEQ)))
    prompts = _sc_gather_prompts(idx.reshape(-1), token_embedding)
    ctx77 = jnp.pad(ctx, ((X_POS, SEQ - X_POS - N_CTX), (0, 0)))
    return _tc_assemble(
        view_label.astype(jnp.int32),
        time_label.astype(jnp.int32),
        prompts.reshape(4, ROWS_PAD, DIM),
        ctx77,
    )


# speedup vs baseline: 5.1300x; 1.0210x over previous
import functools

import jax
import jax.numpy as jnp
from jax import lax
from jax.experimental import pallas as pl
from jax.experimental.pallas import tpu as pltpu
from jax.experimental.pallas import tpu_sc as plsc

SEQ = 77
DIM = 512
N_CTX = 4
X_POS = 5

ROWS_PAD = 80
CHUNK = 16
N_CHUNKS = ROWS_PAD // CHUNK
N_WORKERS = 4 * N_CHUNKS

BS = 32


def _sc_gather_prompts(idx_flat, table):
    info = plsc.get_sparse_core_info()
    nc = info.num_cores
    mesh = plsc.VectorSubcoreMesh(core_axis_name="c", subcore_axis_name="s")

    @functools.partial(
        pl.kernel,
        mesh=mesh,
        out_type=jax.ShapeDtypeStruct((4 * ROWS_PAD, DIM), jnp.float32),
        scratch_types=[
            pltpu.VMEM((CHUNK,), jnp.int32),
            pltpu.VMEM((CHUNK, DIM), jnp.float32),
            pltpu.SemaphoreType.DMA,
        ],
    )
    def k(idx_hbm, table_hbm, out_hbm, idx_v, rows_v, sem):
        wid = lax.axis_index("s") * nc + lax.axis_index("c")

        @pl.when(wid < N_WORKERS)
        def _():
            base = pl.multiple_of(wid * CHUNK, CHUNK)
            pltpu.sync_copy(idx_hbm.at[pl.ds(base, CHUNK)], idx_v)
            pltpu.async_copy(table_hbm.at[idx_v], rows_v, sem).wait()
            pltpu.sync_copy(rows_v, out_hbm.at[pl.ds(base, CHUNK)])

    return k(idx_flat, table)


def _tc_assemble(view, time, prompts, ctx77):
    B = view.shape[0]

    def body(view_ref, time_ref, prompts_ref, ctx_ref, out_ref):
        g = pl.program_id(0)
        row = lax.broadcasted_iota(jnp.int32, (SEQ, DIM), 0)
        is_ctx = (row >= X_POS) & (row < X_POS + N_CTX)
        ctx_rows = ctx_ref[...]
        for j in range(BS):
            b = g * BS + j
            c = view_ref[b] * 2 + time_ref[b]
            out_ref[j] = jnp.where(is_ctx, ctx_rows, prompts_ref[c][0:SEQ, :])

    return pl.pallas_call(
        body,
        grid=(B // BS,),
        in_specs=[
            pl.BlockSpec(memory_space=pltpu.SMEM),
            pl.BlockSpec(memory_space=pltpu.SMEM),
            pl.BlockSpec((4, ROWS_PAD, DIM), lambda g: (0, 0, 0)),
            pl.BlockSpec((SEQ, DIM), lambda g: (0, 0)),
        ],
        out_specs=pl.BlockSpec((BS, SEQ, DIM), lambda g: (g, 0, 0)),
        out_shape=jax.ShapeDtypeStruct((B, SEQ, DIM), jnp.float32),
    )(view, time, prompts, ctx77)


def kernel(label, view_label, time_label, tokenized_table, token_embedding, ctx):
    del label
    idx = jnp.pad(tokenized_table.astype(jnp.int32), ((0, 0), (0, ROWS_PAD - SEQ)))
    prompts = _sc_gather_prompts(idx.reshape(-1), token_embedding)
    ctx77 = jnp.pad(ctx, ((X_POS, SEQ - X_POS - N_CTX), (0, 0)))
    return _tc_assemble(
        view_label.astype(jnp.int32),
        time_label.astype(jnp.int32),
        prompts.reshape(4, ROWS_PAD, DIM),
        ctx77,
    )
